# baseline (speedup 1.0000x reference)
import jax
import jax.numpy as jnp
from jax import lax
from jax.experimental import pallas as pl
from jax.experimental.pallas import tpu as pltpu
from jax.experimental.pallas import tpu_sc as plsc

_DIM = 2048
_N_EXPERTS = 64
_TOPK = 8
_BLOCK = 1024
_TOKENS = 32768
_NCHUNK = 8
_CTOK = _TOKENS // _NCHUNK

_NW = 32
_ROWS_PER_W = _CTOK // _NW
_OUT_PER_W = _ROWS_PER_W * _TOPK


def _probs_block_kernel(x_ref, wt_ref, p_ref):
    x = x_ref[...]
    wt = wt_ref[...]
    scores = lax.dot_general(
        x, wt, dimension_numbers=(((1,), (0,)), ((), ())),
        preferred_element_type=jnp.float32)
    m = jnp.max(scores, axis=-1, keepdims=True)
    e = jnp.exp(scores - m)
    p_ref[...] = e / jnp.sum(e, axis=-1, keepdims=True)


def _tc_probs(x, wt, chunk):
    off = chunk * (_CTOK // _BLOCK)
    grid = (_CTOK // _BLOCK,)
    return pl.pallas_call(
        _probs_block_kernel,
        grid=grid,
        in_specs=[
            pl.BlockSpec((_BLOCK, _DIM), lambda i: (i + off, 0)),
            pl.BlockSpec((_DIM, _N_EXPERTS), lambda i: (0, 0)),
        ],
        out_specs=pl.BlockSpec((_BLOCK, _N_EXPERTS), lambda i: (i, 0)),
        out_shape=jax.ShapeDtypeStruct((_CTOK, _N_EXPERTS), jnp.float32),
    )(x, wt)


def _merge_top16(ka, va, kb, vb):
    rk = lax.rev(kb, (0,))
    rv = lax.rev(vb, (0,))
    c = ka >= rk
    mk = jnp.where(c, ka, rk)
    mv = jnp.where(c, va, rv)
    return plsc.sort_key_val(mk, mv, descending=True)


def _sc_topk_body(probs_hbm, w_hbm, i_hbm, slab_v, w_v, i_v, sem):
    wid = lax.axis_index("s") * 2 + lax.axis_index("c")
    pltpu.async_copy(probs_hbm.at[pl.ds(wid * _ROWS_PER_W, _ROWS_PER_W), :],
                     slab_v, sem).wait()

    iota = lax.iota(jnp.int32, 16)
    m8 = iota < _TOPK
    rowsel = [iota + 16 * j for j in range(4)]

    @plsc.parallel_loop(0, _ROWS_PER_W, 1, unroll=4)
    def _(t):
        srt = [
            plsc.sort_key_val(slab_v[t, pl.ds(16 * j, 16)], rowsel[j],
                              descending=True)
            for j in range(4)
        ]
        k01, v01 = _merge_top16(*srt[0], *srt[1])
        k23, v23 = _merge_top16(*srt[2], *srt[3])
        kf, vf = _merge_top16(k01, v01, k23, v23)
        plsc.store_compressed(w_v.at[pl.ds(t * _TOPK, 16)], kf, mask=m8)
        plsc.store_compressed(i_v.at[pl.ds(t * _TOPK, 16)], vf, mask=m8)

    pltpu.sync_copy(w_v.at[pl.ds(0, _OUT_PER_W)],
                    w_hbm.at[pl.ds(wid * _OUT_PER_W, _OUT_PER_W)])
    pltpu.sync_copy(i_v.at[pl.ds(0, _OUT_PER_W)],
                    i_hbm.at[pl.ds(wid * _OUT_PER_W, _OUT_PER_W)])


def _sc_topk():
    return pl.kernel(
        _sc_topk_body,
        out_type=[
            jax.ShapeDtypeStruct((_CTOK * _TOPK,), jnp.float32),
            jax.ShapeDtypeStruct((_CTOK * _TOPK,), jnp.int32),
        ],
        mesh=plsc.VectorSubcoreMesh(core_axis_name="c", subcore_axis_name="s"),
        compiler_params=pltpu.CompilerParams(needs_layout_passes=False,
                                             skip_device_barrier=True),
        scratch_types=[
            pltpu.VMEM((_ROWS_PER_W, _N_EXPERTS), jnp.float32),
            pltpu.VMEM((_OUT_PER_W + 16,), jnp.float32),
            pltpu.VMEM((_OUT_PER_W + 16,), jnp.int32),
            pltpu.SemaphoreType.DMA,
        ],
    )


def kernel(x, weight):
    wt = weight.T
    sc = _sc_topk()
    ws, inds = [], []
    for c in range(_NCHUNK):
        probs = _tc_probs(x, wt, c)
        w_flat, i_flat = sc(probs)
        ws.append(w_flat.reshape(_CTOK, _TOPK))
        inds.append(i_flat.reshape(_CTOK, _TOPK))
    return (jnp.concatenate(ws, axis=0), jnp.concatenate(inds, axis=0))


# speedup vs baseline: 1.0366x; 1.0366x over previous
import jax
import jax.numpy as jnp
from jax import lax
from jax.experimental import pallas as pl
from jax.experimental.pallas import tpu as pltpu
from jax.experimental.pallas import tpu_sc as plsc

_DIM = 2048
_N_EXPERTS = 64
_TOPK = 8
_BLOCK = 1024
_TOKENS = 32768
_CHUNKS = (12288, 12288, 6144, 2048)

_NW = 32


def _probs_block_kernel(x_ref, wt_ref, p_ref):
    x = x_ref[...]
    wt = wt_ref[...]
    scores = lax.dot_general(
        x, wt, dimension_numbers=(((1,), (0,)), ((), ())),
        preferred_element_type=jnp.float32)
    m = jnp.max(scores, axis=-1, keepdims=True)
    e = jnp.exp(scores - m)
    p_ref[...] = e / jnp.sum(e, axis=-1, keepdims=True)


def _tc_probs(x, wt, tok0, ctok):
    off = tok0 // _BLOCK
    grid = (ctok // _BLOCK,)
    return pl.pallas_call(
        _probs_block_kernel,
        grid=grid,
        in_specs=[
            pl.BlockSpec((_BLOCK, _DIM), lambda i: (i + off, 0)),
            pl.BlockSpec((_DIM, _N_EXPERTS), lambda i: (0, 0)),
        ],
        out_specs=pl.BlockSpec((_BLOCK, _N_EXPERTS), lambda i: (i, 0)),
        out_shape=jax.ShapeDtypeStruct((ctok, _N_EXPERTS), jnp.float32),
    )(x, wt)


def _merge_top16(ka, va, kb, vb):
    rk = lax.rev(kb, (0,))
    rv = lax.rev(vb, (0,))
    c = ka >= rk
    mk = jnp.where(c, ka, rk)
    mv = jnp.where(c, va, rv)
    return plsc.sort_key_val(mk, mv, descending=True)


def _make_sc_topk_body(rows_per_w):
    out_per_w = rows_per_w * _TOPK

    def _sc_topk_body(probs_hbm, w_hbm, i_hbm, slab_v, w_v, i_v, sem):
        wid = lax.axis_index("s") * 2 + lax.axis_index("c")
        pltpu.async_copy(probs_hbm.at[pl.ds(wid * rows_per_w, rows_per_w), :],
                         slab_v, sem).wait()

        iota = lax.iota(jnp.int32, 16)
        m8 = iota < _TOPK
        rowsel = [iota + 16 * j for j in range(4)]

        @plsc.parallel_loop(0, rows_per_w, 1, unroll=4)
        def _(t):
            srt = [
                plsc.sort_key_val(slab_v[t, pl.ds(16 * j, 16)], rowsel[j],
                                  descending=True)
                for j in range(4)
            ]
            k01, v01 = _merge_top16(*srt[0], *srt[1])
            k23, v23 = _merge_top16(*srt[2], *srt[3])
            kf, vf = _merge_top16(k01, v01, k23, v23)
            plsc.store_compressed(w_v.at[pl.ds(t * _TOPK, 16)], kf, mask=m8)
            plsc.store_compressed(i_v.at[pl.ds(t * _TOPK, 16)], vf, mask=m8)

        pltpu.sync_copy(w_v.at[pl.ds(0, out_per_w)],
                        w_hbm.at[pl.ds(wid * out_per_w, out_per_w)])
        pltpu.sync_copy(i_v.at[pl.ds(0, out_per_w)],
                        i_hbm.at[pl.ds(wid * out_per_w, out_per_w)])

    return _sc_topk_body


def _sc_topk(ctok):
    rows_per_w = ctok // _NW
    return pl.kernel(
        _make_sc_topk_body(rows_per_w),
        out_type=[
            jax.ShapeDtypeStruct((ctok * _TOPK,), jnp.float32),
            jax.ShapeDtypeStruct((ctok * _TOPK,), jnp.int32),
        ],
        mesh=plsc.VectorSubcoreMesh(core_axis_name="c", subcore_axis_name="s"),
        compiler_params=pltpu.CompilerParams(needs_layout_passes=False,
                                             skip_device_barrier=True),
        scratch_types=[
            pltpu.VMEM((rows_per_w, _N_EXPERTS), jnp.float32),
            pltpu.VMEM((rows_per_w * _TOPK + 16,), jnp.float32),
            pltpu.VMEM((rows_per_w * _TOPK + 16,), jnp.int32),
            pltpu.SemaphoreType.DMA,
        ],
    )


def kernel(x, weight):
    wt = weight.T
    ws, inds = [], []
    tok0 = 0
    for ctok in _CHUNKS:
        probs = _tc_probs(x, wt, tok0, ctok)
        w_flat, i_flat = _sc_topk(ctok)(probs)
        ws.append(w_flat.reshape(ctok, _TOPK))
        inds.append(i_flat.reshape(ctok, _TOPK))
        tok0 += ctok
    return (jnp.concatenate(ws, axis=0), jnp.concatenate(inds, axis=0))


# speedup vs baseline: 1.0883x; 1.0499x over previous
import jax
import jax.numpy as jnp
from jax import lax
from jax.experimental import pallas as pl
from jax.experimental.pallas import tpu as pltpu
from jax.experimental.pallas import tpu_sc as plsc

_DIM = 2048
_N_EXPERTS = 64
_TOPK = 8
_BLOCK = 1024
_TOKENS = 32768
_CHUNKS = (8192, 8192, 8192, 8192)

_NW = 32


def _probs_block_kernel(x_ref, wt_ref, p_ref):
    x = x_ref[...]
    wt = wt_ref[...]
    scores = lax.dot_general(
        x, wt, dimension_numbers=(((1,), (0,)), ((), ())),
        preferred_element_type=jnp.float32)
    m = jnp.max(scores, axis=-1, keepdims=True)
    e = jnp.exp(scores - m)
    p_ref[...] = e / jnp.sum(e, axis=-1, keepdims=True)


def _tc_probs(x, wt, tok0, ctok):
    off = tok0 // _BLOCK
    grid = (ctok // _BLOCK,)
    return pl.pallas_call(
        _probs_block_kernel,
        grid=grid,
        in_specs=[
            pl.BlockSpec((_BLOCK, _DIM), lambda i: (i + off, 0)),
            pl.BlockSpec((_DIM, _N_EXPERTS), lambda i: (0, 0)),
        ],
        out_specs=pl.BlockSpec((_BLOCK, _N_EXPERTS), lambda i: (i, 0)),
        out_shape=jax.ShapeDtypeStruct((ctok, _N_EXPERTS), jnp.float32),
    )(x, wt)


def _merge_top16(ka, va, kb, vb):
    rk = lax.rev(kb, (0,))
    rv = lax.rev(vb, (0,))
    c = ka >= rk
    mk = jnp.where(c, ka, rk)
    mv = jnp.where(c, va, rv)
    return plsc.sort_key_val(mk, mv, descending=True)


def _make_sc_topk_body(rows_per_w):
    out_per_w = rows_per_w * _TOPK

    def _sc_topk_body(probs_hbm, w_hbm, i_hbm, slab_v, w_v, i_v, sem):
        wid = lax.axis_index("s") * 2 + lax.axis_index("c")
        pltpu.async_copy(probs_hbm.at[pl.ds(wid * rows_per_w, rows_per_w), :],
                         slab_v, sem).wait()

        iota = lax.iota(jnp.int32, 16)
        m8 = iota < _TOPK
        rowsel = [iota + 16 * j for j in range(4)]

        @plsc.parallel_loop(0, rows_per_w, 1, unroll=4)
        def _(t):
            srt = [
                plsc.sort_key_val(slab_v[t, pl.ds(16 * j, 16)], rowsel[j],
                                  descending=True)
                for j in range(4)
            ]
            k01, v01 = _merge_top16(*srt[0], *srt[1])
            k23, v23 = _merge_top16(*srt[2], *srt[3])
            kf, vf = _merge_top16(k01, v01, k23, v23)
            plsc.store_compressed(w_v.at[pl.ds(t * _TOPK, 16)], kf, mask=m8)
            plsc.store_compressed(i_v.at[pl.ds(t * _TOPK, 16)], vf, mask=m8)

        pltpu.sync_copy(w_v.at[pl.ds(0, out_per_w)],
                        w_hbm.at[pl.ds(wid * out_per_w, out_per_w)])
        pltpu.sync_copy(i_v.at[pl.ds(0, out_per_w)],
                        i_hbm.at[pl.ds(wid * out_per_w, out_per_w)])

    return _sc_topk_body


def _sc_topk(ctok):
    rows_per_w = ctok // _NW
    return pl.kernel(
        _make_sc_topk_body(rows_per_w),
        out_type=[
            jax.ShapeDtypeStruct((ctok * _TOPK,), jnp.float32),
            jax.ShapeDtypeStruct((ctok * _TOPK,), jnp.int32),
        ],
        mesh=plsc.VectorSubcoreMesh(core_axis_name="c", subcore_axis_name="s"),
        compiler_params=pltpu.CompilerParams(needs_layout_passes=False,
                                             skip_device_barrier=True),
        scratch_types=[
            pltpu.VMEM((rows_per_w, _N_EXPERTS), jnp.float32),
            pltpu.VMEM((rows_per_w * _TOPK + 16,), jnp.float32),
            pltpu.VMEM((rows_per_w * _TOPK + 16,), jnp.int32),
            pltpu.SemaphoreType.DMA,
        ],
    )


def kernel(x, weight):
    wt = weight.T
    ws, inds = [], []
    tok0 = 0
    for ctok in _CHUNKS:
        probs = _tc_probs(x, wt, tok0, ctok)
        w_flat, i_flat = _sc_topk(ctok)(probs)
        ws.append(w_flat.reshape(ctok, _TOPK))
        inds.append(i_flat.reshape(ctok, _TOPK))
        tok0 += ctok
    return (jnp.concatenate(ws, axis=0), jnp.concatenate(inds, axis=0))


# speedup vs baseline: 1.0896x; 1.0012x over previous
import jax
import jax.numpy as jnp
from jax import lax
from jax.experimental import pallas as pl
from jax.experimental.pallas import tpu as pltpu
from jax.experimental.pallas import tpu_sc as plsc

_DIM = 2048
_N_EXPERTS = 64
_TOPK = 8
_BLOCK = 1024
_TOKENS = 32768
_CHUNKS = (8192, 8192, 8192, 8192)

_NW = 32


def _probs_block_kernel(x_ref, wt_ref, p_ref):
    x = x_ref[...]
    wt = wt_ref[...]
    scores = lax.dot_general(
        x, wt, dimension_numbers=(((1,), (0,)), ((), ())),
        preferred_element_type=jnp.float32)
    m = jnp.max(scores, axis=-1, keepdims=True)
    e = jnp.exp(scores - m)
    p_ref[...] = e / jnp.sum(e, axis=-1, keepdims=True)


def _tc_probs(x, wt, tok0, ctok):
    off = tok0 // _BLOCK
    grid = (ctok // _BLOCK,)
    return pl.pallas_call(
        _probs_block_kernel,
        grid=grid,
        in_specs=[
            pl.BlockSpec((_BLOCK, _DIM), lambda i: (i + off, 0)),
            pl.BlockSpec((_DIM, _N_EXPERTS), lambda i: (0, 0)),
        ],
        out_specs=pl.BlockSpec((_BLOCK, _N_EXPERTS), lambda i: (i, 0)),
        out_shape=jax.ShapeDtypeStruct((ctok, _N_EXPERTS), jnp.float32),
    )(x, wt)


def _merge_top16(ka, va, kb, vb):
    rk = lax.rev(kb, (0,))
    rv = lax.rev(vb, (0,))
    c = ka >= rk
    mk = jnp.where(c, ka, rk)
    mv = jnp.where(c, va, rv)
    return plsc.sort_key_val(mk, mv, descending=True)


def _make_sc_topk_body(rows_per_w):
    out_per_w = rows_per_w * _TOPK

    def _sc_topk_body(probs_hbm, w_hbm, i_hbm, slab_v, w_v, i_v, sem):
        wid = lax.axis_index("s") * 2 + lax.axis_index("c")
        pltpu.async_copy(probs_hbm.at[pl.ds(wid * rows_per_w, rows_per_w), :],
                         slab_v, sem).wait()

        iota = lax.iota(jnp.int32, 16)
        m8 = iota < _TOPK
        rowsel = [iota + 16 * j for j in range(4)]

        @plsc.parallel_loop(0, rows_per_w, 1, unroll=4)
        def _(t):
            srt = [
                plsc.sort_key_val(slab_v[t, pl.ds(16 * j, 16)], rowsel[j],
                                  descending=True)
                for j in range(4)
            ]
            k01, v01 = _merge_top16(*srt[0], *srt[1])
            k23, v23 = _merge_top16(*srt[2], *srt[3])
            kf, vf = _merge_top16(k01, v01, k23, v23)
            plsc.store_compressed(w_v.at[pl.ds(t * _TOPK, 16)], kf, mask=m8)
            plsc.store_compressed(i_v.at[pl.ds(t * _TOPK, 16)], vf, mask=m8)

        pltpu.sync_copy(w_v.at[pl.ds(0, out_per_w)],
                        w_hbm.at[pl.ds(wid * out_per_w, out_per_w)])
        pltpu.sync_copy(i_v.at[pl.ds(0, out_per_w)],
                        i_hbm.at[pl.ds(wid * out_per_w, out_per_w)])

    return _sc_topk_body


def _sc_topk(ctok):
    rows_per_w = ctok // _NW
    return pl.kernel(
        _make_sc_topk_body(rows_per_w),
        out_type=[
            jax.ShapeDtypeStruct((ctok * _TOPK,), jnp.float32),
            jax.ShapeDtypeStruct((ctok * _TOPK,), jnp.int32),
        ],
        mesh=plsc.VectorSubcoreMesh(core_axis_name="c", subcore_axis_name="s"),
        compiler_params=pltpu.CompilerParams(needs_layout_passes=False),
        scratch_types=[
            pltpu.VMEM((rows_per_w, _N_EXPERTS), jnp.float32),
            pltpu.VMEM((rows_per_w * _TOPK + 16,), jnp.float32),
            pltpu.VMEM((rows_per_w * _TOPK + 16,), jnp.int32),
            pltpu.SemaphoreType.DMA,
        ],
    )


def kernel(x, weight):
    wt = weight.T
    ws, inds = [], []
    tok0 = 0
    for ctok in _CHUNKS:
        probs = _tc_probs(x, wt, tok0, ctok)
        w_flat, i_flat = _sc_topk(ctok)(probs)
        ws.append(w_flat.reshape(ctok, _TOPK))
        inds.append(i_flat.reshape(ctok, _TOPK))
        tok0 += ctok
    return (jnp.concatenate(ws, axis=0), jnp.concatenate(inds, axis=0))
